# baseline (speedup 1.0000x reference)
import jax
import jax.numpy as jnp
from jax import lax
from jax.experimental import pallas as pl
from jax.experimental.pallas import tpu as pltpu
from jax.experimental.pallas import tpu_sc as plsc

N = 1600000
NUM_GRAPHS = 4096
IN_DIM = 3
HID = 32


_BL = 16384


def _mlp_body(x_ref, w1t_ref, b1_ref, w2_ref, b2_ref, o_ref):
    x = x_ref[...]
    h = jnp.dot(w1t_ref[...], x, preferred_element_type=jnp.float32)
    h = h + b1_ref[...].reshape(HID, 1)
    h = h * jax.nn.sigmoid(h)
    e = jnp.sum(h * w2_ref[...].reshape(HID, 1), axis=0)
    o_ref[...] = e + b2_ref[...]


def _node_energy(pos_t, W1T, b1, W2, b2):
    return pl.pallas_call(
        _mlp_body,
        grid=(pl.cdiv(N, _BL),),
        in_specs=[
            pl.BlockSpec((IN_DIM, _BL), lambda i: (0, i)),
            pl.BlockSpec((HID, IN_DIM), lambda i: (0, 0)),
            pl.BlockSpec((HID,), lambda i: (0,)),
            pl.BlockSpec((HID, 1), lambda i: (0, 0)),
            pl.BlockSpec((1,), lambda i: (0,)),
        ],
        out_specs=pl.BlockSpec((_BL,), lambda i: (i,)),
        out_shape=jax.ShapeDtypeStruct((N,), jnp.float32),
    )(pos_t, W1T, b1, W2, b2)


_NS = 16
_ROWS_PER_TILE = N // _NS
_CHUNK = 4000
_G_ROWS = NUM_GRAPHS // 128


def _seg_body(e_hbm, i_hbm, out_hbm, ev, iv, acc, acc2, rowidx, shared):
    sid = lax.axis_index("s")
    base = sid * _ROWS_PER_TILE

    def _zero(j, _):
        acc[pl.ds(j * 16, 16)] = jnp.zeros((16,), jnp.float32)
        return _

    lax.fori_loop(0, NUM_GRAPHS // 16, _zero, None)

    rowidx[pl.ds(0, 16)] = lax.iota(jnp.int32, 16)
    rowidx[pl.ds(16, 16)] = lax.iota(jnp.int32, 16) + 16

    def _zero2(j, _):
        acc2[j >> 3, pl.ds((j & 7) * 16, 16)] = jnp.zeros((16,), jnp.float32)
        return _

    lax.fori_loop(0, _G_ROWS * 8, _zero2, None)

    @pl.when(sid == 0)
    def _():
        pltpu.sync_copy(acc2, shared)

    def _chunk(ci, _):
        off = base + ci * _CHUNK
        pltpu.sync_copy(e_hbm.at[pl.ds(off, _CHUNK)], ev)
        pltpu.sync_copy(i_hbm.at[pl.ds(off, _CHUNK)], iv)

        def _inner(j, _):
            g = iv[pl.ds(j * 16, 16)]
            vals = ev[pl.ds(j * 16, 16)]
            plsc.addupdate_scatter(acc, [g], vals)
            return _

        lax.fori_loop(0, _CHUNK // 16, _inner, None)
        return _

    lax.fori_loop(0, _ROWS_PER_TILE // _CHUNK, _chunk, None)

    def _stage(j, _):
        acc2[j >> 3, pl.ds((j & 7) * 16, 16)] = acc[pl.ds(j * 16, 16)]
        return _

    lax.fori_loop(0, NUM_GRAPHS // 16, _stage, None)
    plsc.subcore_barrier()
    pltpu.sync_copy(acc2, shared.at[rowidx], add=True)
    plsc.subcore_barrier()

    @pl.when(sid == 0)
    def _():
        pltpu.sync_copy(shared, out_hbm)


def _segment_sum(node_energy, batch32):
    mesh = plsc.VectorSubcoreMesh(
        core_axis_name="c", subcore_axis_name="s", num_cores=1
    )
    seg = pl.kernel(
        _seg_body,
        out_type=jax.ShapeDtypeStruct((_G_ROWS, 128), jnp.float32),
        mesh=mesh,
        scratch_types=[
            pltpu.VMEM((_CHUNK,), jnp.float32),
            pltpu.VMEM((_CHUNK,), jnp.int32),
            pltpu.VMEM((NUM_GRAPHS,), jnp.float32),
            pltpu.VMEM((_G_ROWS, 128), jnp.float32),
            pltpu.VMEM((2 * _NS,), jnp.int32),
            pltpu.VMEM_SHARED((_G_ROWS, 128), jnp.float32),
        ],
        compiler_params=pltpu.CompilerParams(needs_layout_passes=False),
    )
    return seg(node_energy, batch32)


@jax.jit
def kernel(positions, batch, W1, b1, W2, b2):
    batch32 = batch.astype(jnp.int32)
    pos_t = positions.T
    node_energy = _node_energy(pos_t, W1.T, b1, W2, b2)
    energy = _segment_sum(node_energy, batch32)
    return energy.reshape(NUM_GRAPHS)


# speedup vs baseline: 1.5316x; 1.5316x over previous
import jax
import jax.numpy as jnp
from jax import lax
from jax.experimental import pallas as pl
from jax.experimental.pallas import tpu as pltpu
from jax.experimental.pallas import tpu_sc as plsc

N = 1600000
NUM_GRAPHS = 4096
IN_DIM = 3
HID = 32


_BL = 16384


def _mlp_body(x_ref, w1t_ref, b1_ref, w2_ref, b2_ref, o_ref):
    x = x_ref[...]
    h = jnp.dot(w1t_ref[...], x, preferred_element_type=jnp.float32)
    h = h + b1_ref[...].reshape(HID, 1)
    h = h * jax.nn.sigmoid(h)
    e = jnp.sum(h * w2_ref[...].reshape(HID, 1), axis=0)
    o_ref[...] = e + b2_ref[...]


def _node_energy(pos_t, W1T, b1, W2, b2):
    return pl.pallas_call(
        _mlp_body,
        grid=(pl.cdiv(N, _BL),),
        in_specs=[
            pl.BlockSpec((IN_DIM, _BL), lambda i: (0, i)),
            pl.BlockSpec((HID, IN_DIM), lambda i: (0, 0)),
            pl.BlockSpec((HID,), lambda i: (0,)),
            pl.BlockSpec((HID, 1), lambda i: (0, 0)),
            pl.BlockSpec((1,), lambda i: (0,)),
        ],
        out_specs=pl.BlockSpec((_BL,), lambda i: (i,)),
        out_shape=jax.ShapeDtypeStruct((N,), jnp.float32),
    )(pos_t, W1T, b1, W2, b2)


_NC = 2
_NS = 16
_NW = _NC * _NS
_ROWS_PER_TILE = N // _NW
_CHUNK = 10000
_NCHUNK = _ROWS_PER_TILE // _CHUNK


def _seg_body(e_hbm, i_hbm, out_hbm, ev0, iv0, ev1, iv1, acc, sems):
    cid = lax.axis_index("c")
    sid = lax.axis_index("s")
    wid = cid * _NS + sid
    base = wid * _ROWS_PER_TILE
    evs, ivs = (ev0, ev1), (iv0, iv1)

    def _start(c, b):
        off = base + c * _CHUNK
        pltpu.async_copy(e_hbm.at[pl.ds(off, _CHUNK)], evs[b], sems.at[2 * b])
        pltpu.async_copy(i_hbm.at[pl.ds(off, _CHUNK)], ivs[b], sems.at[2 * b + 1])

    def _wait(b):
        pltpu.make_async_copy(
            e_hbm.at[pl.ds(0, _CHUNK)], evs[b], sems.at[2 * b]).wait()
        pltpu.make_async_copy(
            i_hbm.at[pl.ds(0, _CHUNK)], ivs[b], sems.at[2 * b + 1]).wait()

    _start(0, 0)
    if _NCHUNK > 1:
        _start(1, 1)

    def _zero(j, _):
        acc[pl.ds(j * 16, 16)] = jnp.zeros((16,), jnp.float32)
        return _

    lax.fori_loop(0, NUM_GRAPHS // 16, _zero, None, unroll=8)

    for c in range(_NCHUNK):
        b = c & 1
        _wait(b)
        ev, iv = evs[b], ivs[b]

        def _inner(j, _):
            g = iv[pl.ds(j * 16, 16)]
            vals = ev[pl.ds(j * 16, 16)]
            plsc.addupdate_scatter(acc, [g], vals)
            return _

        lax.fori_loop(0, _CHUNK // 16, _inner, None, unroll=8)
        if c + 2 < _NCHUNK:
            _start(c + 2, b)

    pltpu.sync_copy(acc, out_hbm.at[wid])


def _segment_sum(node_energy, batch32):
    mesh = plsc.VectorSubcoreMesh(core_axis_name="c", subcore_axis_name="s")
    seg = pl.kernel(
        _seg_body,
        out_type=jax.ShapeDtypeStruct((_NW, NUM_GRAPHS), jnp.float32),
        mesh=mesh,
        scratch_types=[
            pltpu.VMEM((_CHUNK,), jnp.float32),
            pltpu.VMEM((_CHUNK,), jnp.int32),
            pltpu.VMEM((_CHUNK,), jnp.float32),
            pltpu.VMEM((_CHUNK,), jnp.int32),
            pltpu.VMEM((NUM_GRAPHS,), jnp.float32),
            pltpu.SemaphoreType.DMA((4,)),
        ],
        compiler_params=pltpu.CompilerParams(needs_layout_passes=False),
    )
    return seg(node_energy, batch32)


def _comb_body(a_ref, o_ref):
    o_ref[...] = jnp.sum(a_ref[...], axis=0)


def _combine(parts):
    return pl.pallas_call(
        _comb_body,
        out_shape=jax.ShapeDtypeStruct((NUM_GRAPHS // 128, 128), jnp.float32),
    )(parts)


@jax.jit
def kernel(positions, batch, W1, b1, W2, b2):
    batch32 = batch.astype(jnp.int32)
    pos_t = positions.T
    node_energy = _node_energy(pos_t, W1.T, b1, W2, b2)
    parts = _segment_sum(node_energy, batch32)
    parts3 = parts.reshape(_NW, NUM_GRAPHS // 128, 128)
    return _combine(parts3).reshape(NUM_GRAPHS)


# speedup vs baseline: 2.0383x; 1.3309x over previous
import jax
import jax.numpy as jnp
from jax import lax
from jax.experimental import pallas as pl
from jax.experimental.pallas import tpu as pltpu
from jax.experimental.pallas import tpu_sc as plsc

N = 1600000
NUM_GRAPHS = 4096
IN_DIM = 3
HID = 32


_BL = 16384


def _mlp_body(x_ref, w1t_ref, b1_ref, w2_ref, b2_ref, o_ref):
    x = x_ref[...]
    h = jnp.dot(w1t_ref[...], x, preferred_element_type=jnp.float32)
    h = h + b1_ref[...].reshape(HID, 1)
    h = h * jax.nn.sigmoid(h)
    e = jnp.sum(h * w2_ref[...].reshape(HID, 1), axis=0)
    o_ref[...] = e + b2_ref[...]


def _node_energy(pos_t, W1T, b1, W2, b2):
    return pl.pallas_call(
        _mlp_body,
        grid=(pl.cdiv(N, _BL),),
        in_specs=[
            pl.BlockSpec((IN_DIM, _BL), lambda i: (0, i)),
            pl.BlockSpec((HID, IN_DIM), lambda i: (0, 0)),
            pl.BlockSpec((HID,), lambda i: (0,)),
            pl.BlockSpec((HID, 1), lambda i: (0, 0)),
            pl.BlockSpec((1,), lambda i: (0,)),
        ],
        out_specs=pl.BlockSpec((_BL,), lambda i: (i,)),
        out_shape=jax.ShapeDtypeStruct((N,), jnp.float32),
    )(pos_t, W1T, b1, W2, b2)


_NC = 2
_NS = 16
_NW = _NC * _NS
_ROWS_PER_TILE = N // _NW
_CHUNK = 10000
_NCHUNK = _ROWS_PER_TILE // _CHUNK
_STRIDE = _CHUNK // 16


def _seg_body(e_hbm, i_hbm, out_hbm, ev0, iv0, ev1, iv1, acc, sems):
    cid = lax.axis_index("c")
    sid = lax.axis_index("s")
    wid = cid * _NS + sid
    base = wid * _ROWS_PER_TILE
    evs, ivs = (ev0, ev1), (iv0, iv1)

    def _start(c, b):
        off = base + c * _CHUNK
        pltpu.async_copy(e_hbm.at[pl.ds(off, _CHUNK)], evs[b], sems.at[2 * b])
        pltpu.async_copy(i_hbm.at[pl.ds(off, _CHUNK)], ivs[b], sems.at[2 * b + 1])

    def _wait(b):
        pltpu.make_async_copy(
            e_hbm.at[pl.ds(0, _CHUNK)], evs[b], sems.at[2 * b]).wait()
        pltpu.make_async_copy(
            i_hbm.at[pl.ds(0, _CHUNK)], ivs[b], sems.at[2 * b + 1]).wait()

    _start(0, 0)
    if _NCHUNK > 1:
        _start(1, 1)

    def _zero(j, _):
        acc[pl.ds(j * 16, 16)] = jnp.zeros((16,), jnp.float32)
        return _

    lax.fori_loop(0, NUM_GRAPHS // 16, _zero, None, unroll=8)

    stride_iota = lax.iota(jnp.int32, 16) * _STRIDE

    for c in range(_NCHUNK):
        b = c & 1
        _wait(b)
        ev, iv = evs[b], ivs[b]

        def _inner(j, _):
            idxv = stride_iota + j
            g = plsc.load_gather(iv, [idxv])
            vals = plsc.load_gather(ev, [idxv])
            plsc.addupdate_scatter(acc, [g], vals)
            return _

        lax.fori_loop(0, _STRIDE, _inner, None, unroll=8)
        if c + 2 < _NCHUNK:
            _start(c + 2, b)

    pltpu.sync_copy(acc, out_hbm.at[wid])


def _segment_sum(node_energy, batch32):
    mesh = plsc.VectorSubcoreMesh(core_axis_name="c", subcore_axis_name="s")
    seg = pl.kernel(
        _seg_body,
        out_type=jax.ShapeDtypeStruct((_NW, NUM_GRAPHS), jnp.float32),
        mesh=mesh,
        scratch_types=[
            pltpu.VMEM((_CHUNK,), jnp.float32),
            pltpu.VMEM((_CHUNK,), jnp.int32),
            pltpu.VMEM((_CHUNK,), jnp.float32),
            pltpu.VMEM((_CHUNK,), jnp.int32),
            pltpu.VMEM((NUM_GRAPHS,), jnp.float32),
            pltpu.SemaphoreType.DMA((4,)),
        ],
        compiler_params=pltpu.CompilerParams(needs_layout_passes=False),
    )
    return seg(node_energy, batch32)


def _comb_body(a_ref, o_ref):
    o_ref[...] = jnp.sum(a_ref[...], axis=0)


def _combine(parts):
    return pl.pallas_call(
        _comb_body,
        out_shape=jax.ShapeDtypeStruct((NUM_GRAPHS // 128, 128), jnp.float32),
    )(parts)


@jax.jit
def kernel(positions, batch, W1, b1, W2, b2):
    batch32 = batch.astype(jnp.int32)
    pos_t = positions.T
    node_energy = _node_energy(pos_t, W1.T, b1, W2, b2)
    parts = _segment_sum(node_energy, batch32)
    parts3 = parts.reshape(_NW, NUM_GRAPHS // 128, 128)
    return _combine(parts3).reshape(NUM_GRAPHS)
